# baseline (speedup 1.0000x reference)
import functools

import jax
import jax.numpy as jnp
from jax import lax
from jax.experimental import pallas as pl
from jax.experimental.pallas import tpu as pltpu
from jax.experimental.pallas import tpu_sc as plsc

N_NODES = 50000
HALF = 25000
D = 64
N_EDGES = 800000
N_TILES = 16
TILE_ROWS = 393
ROWS2D = TILE_ROWS * N_TILES
EDGES_PAD = ROWS2D * 128
NOUT = TILE_ROWS // 3
ACC_ROWS = 25216
ZROWS = ACC_ROWS // N_TILES
OROWS = 1560
LAST = TILE_ROWS - 1

_mesh = plsc.VectorSubcoreMesh(core_axis_name="c", subcore_axis_name="s")


@functools.partial(
    pl.kernel,
    mesh=_mesh,
    compiler_params=pltpu.CompilerParams(use_tc_tiling_on_sc=False),
    out_type=jax.ShapeDtypeStruct((N_NODES, D), jnp.float32),
    scratch_types=[
        pltpu.VMEM((3, 2, 128), jnp.int32),
        pltpu.VMEM((3, 128), jnp.float32),
        pltpu.VMEM((3, 128), jnp.int32),
        pltpu.VMEM((3, 128, D), jnp.float32),
        pltpu.VMEM_SHARED((ACC_ROWS, D), jnp.float32),
        pltpu.SemaphoreType.DMA,
        pltpu.SemaphoreType.DMA,
        pltpu.SemaphoreType.DMA,
        pltpu.SemaphoreType.DMA,
        pltpu.SemaphoreType.DMA,
        pltpu.SemaphoreType.DMA,
        pltpu.SemaphoreType.DMA,
        pltpu.SemaphoreType.DMA,
        pltpu.SemaphoreType.DMA,
    ],
)
def _spmm_kernel(tab, epack, val2, out, idx3, valv, midx, rows, acc,
                 isem0, isem1, isem2, gsem0, gsem1, gsem2,
                 ssem0, ssem1, ssem2):
    isem = (isem0, isem1, isem2)
    gsem = (gsem0, gsem1, gsem2)
    ssem = (ssem0, ssem1, ssem2)
    c = lax.axis_index("c")
    s = lax.axis_index("s")
    off = c * HALF

    def zrow(r, carry):
        for db in range(D // 16):
            rows[0, r, pl.ds(db * 16, 16)] = jnp.zeros((16,), jnp.float32)
        return carry
    lax.fori_loop(0, 128, zrow, 0)
    zb = s * ZROWS

    def zacc(t, carry):
        pltpu.sync_copy(rows.at[0], acc.at[pl.ds(zb + t * 128, 128)])
        return carry
    lax.fori_loop(0, ZROWS // 128, zacc, 0)
    pltpu.sync_copy(rows.at[0, pl.ds(0, ZROWS % 128)],
                    acc.at[pl.ds(zb + (ZROWS // 128) * 128, ZROWS % 128)])
    plsc.subcore_barrier()

    rbase = s * TILE_ROWS

    for b in range(3):
        pltpu.async_copy(epack.at[rbase + b], idx3.at[b], isem[b])
        pltpu.async_copy(val2.at[rbase + b], valv.at[b], isem[b])
    pltpu.make_async_copy(epack.at[rbase], idx3.at[0], isem[0]).wait()
    pltpu.make_async_copy(val2.at[rbase], valv.at[0], isem[0]).wait()
    pltpu.async_copy(tab.at[idx3.at[0, 0]], rows.at[0], gsem[0])

    def step(g, carry):
        for b in range(3):
            b1 = (b + 1) % 3
            i = g * 3 + b
            pltpu.make_async_copy(tab.at[pl.ds(0, 128)], rows.at[b],
                                  gsem[b]).wait()

            def _issue_gather():
                pltpu.make_async_copy(
                    epack.at[rbase], idx3.at[b1], isem[b1]).wait()
                pltpu.make_async_copy(
                    val2.at[rbase], valv.at[b1], isem[b1]).wait()

                def _wait_scatter():
                    pltpu.make_async_copy(tab.at[pl.ds(0, 128)],
                                          rows.at[b1], ssem[b1]).wait()
                if b < 2:
                    pl.when(g > 0)(_wait_scatter)
                else:
                    _wait_scatter()
                pltpu.async_copy(tab.at[idx3.at[b1, 0]], rows.at[b1],
                                 gsem[b1])
            if b < 2:
                _issue_gather()
            else:
                pl.when(g < NOUT - 1)(_issue_gather)

            for i8 in range(8):
                dv = idx3[b, 1, pl.ds(i8 * 16, 16)]
                loc = dv - off
                okm = (loc >= 0) & (loc < HALF)
                trash = lax.iota(jnp.int32, 16) + (HALF + 8 + i8 * 16)
                midx[b, pl.ds(i8 * 16, 16)] = jnp.where(okm, loc, trash)

            def sgrp(i8, carry2):
                vv = valv[b, pl.ds(i8 * 16, 16)]
                rr = i8 * 16
                for k in range(16):
                    sv = vv[k]
                    for db in range(D // 16):
                        rows[b, rr + k, pl.ds(db * 16, 16)] = (
                            rows[b, rr + k, pl.ds(db * 16, 16)] * sv)
                return carry2
            lax.fori_loop(0, 8, sgrp, 0)

            pltpu.async_copy(rows.at[b], acc.at[midx.at[b]], ssem[b],
                             add=True)

            def _stage():
                pltpu.async_copy(epack.at[rbase + i + 3], idx3.at[b],
                                 isem[b])
                pltpu.async_copy(val2.at[rbase + i + 3], valv.at[b],
                                 isem[b])
            pl.when(g < NOUT - 1)(_stage)
        return carry

    lax.fori_loop(0, NOUT, step, 0)
    for b in range(3):
        pltpu.make_async_copy(tab.at[pl.ds(0, 128)], rows.at[b],
                              ssem[b]).wait()
    plsc.subcore_barrier()

    ob = s * OROWS
    pltpu.sync_copy(acc.at[pl.ds(ob, OROWS)], out.at[pl.ds(off + ob, OROWS)])

    @pl.when(s == 0)
    def _tail():
        rem = HALF - N_TILES * OROWS
        pltpu.sync_copy(acc.at[pl.ds(N_TILES * OROWS, rem)],
                        out.at[pl.ds(off + N_TILES * OROWS, rem)])


def _mean_body(a, b, cc, d, o):
    o[...] = (a[...] + b[...] + cc[...] + d[...]) * 0.25


def _mean4(e0, e1, e2, e3):
    shaped = [e.reshape(N_NODES * D // 128, 128) for e in (e0, e1, e2, e3)]
    out = pl.pallas_call(
        _mean_body,
        grid=(25,),
        in_specs=[pl.BlockSpec((1000, 128), lambda i: (i, 0))] * 4,
        out_specs=pl.BlockSpec((1000, 128), lambda i: (i, 0)),
        out_shape=jax.ShapeDtypeStruct((N_NODES * D // 128, 128), jnp.float32),
    )(*shaped)
    return out.reshape(N_NODES, D)


def kernel(embeddings, edge_values, edge_index):
    src = edge_index[0].astype(jnp.int32)
    dst = edge_index[1].astype(jnp.int32)
    val = edge_values.astype(jnp.float32)
    pad = EDGES_PAD - N_EDGES
    src = jnp.concatenate([src, jnp.zeros((pad,), jnp.int32)]).reshape(
        ROWS2D, 1, 128)
    dst = jnp.concatenate([dst, jnp.zeros((pad,), jnp.int32)]).reshape(
        ROWS2D, 1, 128)
    val2 = jnp.concatenate([val, jnp.zeros((pad,), jnp.float32)]).reshape(
        ROWS2D, 128)
    epack = jnp.concatenate([src, dst], axis=1)
    e0 = embeddings
    e1 = _spmm_kernel(e0, epack, val2)
    e2 = _spmm_kernel(e1, epack, val2)
    e3 = _spmm_kernel(e2, epack, val2)
    mean = _mean4(e0, e1, e2, e3)
    return mean[:HALF], mean[HALF:]


# speedup vs baseline: 1.6539x; 1.6539x over previous
import functools

import jax
import jax.numpy as jnp
from jax import lax
from jax.experimental import pallas as pl
from jax.experimental.pallas import tpu as pltpu
from jax.experimental.pallas import tpu_sc as plsc

N_NODES = 50000
HALF = 25000
D = 64
N_EDGES = 800000
N_TILES = 16
SUB = 96
TILE_SUB = 524
ROWS3 = TILE_SUB * N_TILES
EDGES_PAD = ROWS3 * SUB
NOUT = TILE_SUB // 4
ACC_ROWS = 25008
ZROWS = ACC_ROWS // N_TILES
OROWS = 1560

_mesh = plsc.VectorSubcoreMesh(core_axis_name="c", subcore_axis_name="s")


@functools.partial(
    pl.kernel,
    mesh=_mesh,
    compiler_params=pltpu.CompilerParams(use_tc_tiling_on_sc=False),
    out_type=jax.ShapeDtypeStruct((N_NODES, D), jnp.float32),
    scratch_types=[
        pltpu.VMEM((4, 2, SUB), jnp.int32),
        pltpu.VMEM((4, SUB), jnp.float32),
        pltpu.VMEM((4, SUB), jnp.int32),
        pltpu.VMEM((4, SUB, D), jnp.float32),
        pltpu.VMEM_SHARED((ACC_ROWS, D), jnp.float32),
        pltpu.SemaphoreType.DMA,
        pltpu.SemaphoreType.DMA,
        pltpu.SemaphoreType.DMA,
        pltpu.SemaphoreType.DMA,
        pltpu.SemaphoreType.DMA,
        pltpu.SemaphoreType.DMA,
        pltpu.SemaphoreType.DMA,
        pltpu.SemaphoreType.DMA,
        pltpu.SemaphoreType.DMA,
        pltpu.SemaphoreType.DMA,
        pltpu.SemaphoreType.DMA,
        pltpu.SemaphoreType.DMA,
    ],
)
def _spmm_kernel(tab, epack, val2, out, idx2, valv, midx, rows, acc,
                 isem0, isem1, isem2, isem3, gsem0, gsem1, gsem2, gsem3,
                 ssem0, ssem1, ssem2, ssem3):
    isem = (isem0, isem1, isem2, isem3)
    gsem = (gsem0, gsem1, gsem2, gsem3)
    ssem = (ssem0, ssem1, ssem2, ssem3)
    c = lax.axis_index("c")
    s = lax.axis_index("s")
    off = c * HALF

    def zrow(r, carry):
        for db in range(D // 16):
            rows[0, r, pl.ds(db * 16, 16)] = jnp.zeros((16,), jnp.float32)
        return carry
    lax.fori_loop(0, SUB, zrow, 0)
    zb = s * ZROWS

    def zacc(t, carry):
        pltpu.sync_copy(rows.at[0], acc.at[pl.ds(zb + t * SUB, SUB)])
        return carry
    lax.fori_loop(0, ZROWS // SUB, zacc, 0)
    pltpu.sync_copy(rows.at[0, pl.ds(0, ZROWS % SUB)],
                    acc.at[pl.ds(zb + (ZROWS // SUB) * SUB, ZROWS % SUB)])
    plsc.subcore_barrier()

    rbase = s * TILE_SUB

    def _stage_idx(row, slot):
        pltpu.async_copy(epack.at[row], idx2.at[slot], isem[slot])
        pltpu.async_copy(val2.at[row], valv.at[slot], isem[slot])

    def _wait_idx(slot):
        pltpu.make_async_copy(epack.at[rbase], idx2.at[slot],
                              isem[slot]).wait()
        pltpu.make_async_copy(val2.at[rbase], valv.at[slot],
                              isem[slot]).wait()

    def _wait_g(slot):
        pltpu.make_async_copy(tab.at[pl.ds(0, SUB)], rows.at[slot],
                              gsem[slot]).wait()

    def _wait_s(slot):
        pltpu.make_async_copy(tab.at[pl.ds(0, SUB)], rows.at[slot],
                              ssem[slot]).wait()

    def _gather(slot):
        pltpu.async_copy(tab.at[idx2.at[slot, 0]], rows.at[slot], gsem[slot])

    for b in range(4):
        _stage_idx(rbase + b, b)
    _wait_idx(0)
    _gather(0)
    _wait_idx(1)
    _gather(1)

    def step(g, carry):
        for b in range(4):
            b2 = (b + 2) % 4
            i = g * 4 + b
            _wait_g(b)

            def _issue_gather():
                _wait_idx(b2)

                def _ws():
                    _wait_s(b2)
                if b < 2:
                    pl.when(g > 0)(_ws)
                else:
                    _ws()
                _gather(b2)
            if b < 2:
                _issue_gather()
            else:
                pl.when(g < NOUT - 1)(_issue_gather)

            for i6 in range(SUB // 16):
                dv = idx2[b, 1, pl.ds(i6 * 16, 16)]
                loc = dv - off
                okm = (loc >= 0) & (loc < HALF)
                trash = (lax.iota(jnp.int32, 16) & 7) + HALF
                midx[b, pl.ds(i6 * 16, 16)] = jnp.where(okm, loc, trash)

            def sgrp(i6, carry2):
                vv = valv[b, pl.ds(i6 * 16, 16)]
                rr = i6 * 16
                for k in range(16):
                    sv = vv[k]
                    for db in range(D // 16):
                        rows[b, rr + k, pl.ds(db * 16, 16)] = (
                            rows[b, rr + k, pl.ds(db * 16, 16)] * sv)
                return carry2
            lax.fori_loop(0, SUB // 16, sgrp, 0)

            pltpu.async_copy(rows.at[b], acc.at[midx.at[b]], ssem[b],
                             add=True)

            def _stage():
                _stage_idx(rbase + i + 4, b)
            pl.when(g < NOUT - 1)(_stage)
        return carry

    lax.fori_loop(0, NOUT, step, 0)
    for b in range(4):
        _wait_s(b)
    plsc.subcore_barrier()

    ob = s * OROWS
    pltpu.sync_copy(acc.at[pl.ds(ob, OROWS)], out.at[pl.ds(off + ob, OROWS)])

    @pl.when(s == 0)
    def _tail():
        rem = HALF - N_TILES * OROWS
        pltpu.sync_copy(acc.at[pl.ds(N_TILES * OROWS, rem)],
                        out.at[pl.ds(off + N_TILES * OROWS, rem)])


def _mean_body(a, b, cc, d, o):
    o[...] = (a[...] + b[...] + cc[...] + d[...]) * 0.25


def _mean4(e0, e1, e2, e3):
    shaped = [e.reshape(N_NODES * D // 128, 128) for e in (e0, e1, e2, e3)]
    out = pl.pallas_call(
        _mean_body,
        grid=(25,),
        in_specs=[pl.BlockSpec((1000, 128), lambda i: (i, 0))] * 4,
        out_specs=pl.BlockSpec((1000, 128), lambda i: (i, 0)),
        out_shape=jax.ShapeDtypeStruct((N_NODES * D // 128, 128), jnp.float32),
    )(*shaped)
    return out.reshape(N_NODES, D)


def kernel(embeddings, edge_values, edge_index):
    src = edge_index[0].astype(jnp.int32)
    dst = edge_index[1].astype(jnp.int32)
    val = edge_values.astype(jnp.float32)
    pad = EDGES_PAD - N_EDGES
    src = jnp.concatenate([src, jnp.zeros((pad,), jnp.int32)]).reshape(
        ROWS3, 1, SUB)
    dst = jnp.concatenate([dst, jnp.zeros((pad,), jnp.int32)]).reshape(
        ROWS3, 1, SUB)
    val2 = jnp.concatenate([val, jnp.zeros((pad,), jnp.float32)]).reshape(
        ROWS3, SUB)
    epack = jnp.concatenate([src, dst], axis=1)
    e0 = embeddings
    e1 = _spmm_kernel(e0, epack, val2)
    e2 = _spmm_kernel(e1, epack, val2)
    e3 = _spmm_kernel(e2, epack, val2)
    mean = _mean4(e0, e1, e2, e3)
    return mean[:HALF], mean[HALF:]
